# baseline (speedup 1.0000x reference)
import functools

import jax
import jax.numpy as jnp
from jax import lax
from jax.experimental import pallas as pl
from jax.experimental.pallas import tpu as pltpu
from jax.experimental.pallas import tpu_sc as plsc

_NC = 2
_NS = 16
_NW = _NC * _NS
_L = 16
_ROW = 128

_W = 256


@functools.lru_cache(maxsize=None)
def _make_pad(vocab, dim):
    nblk, tail = divmod(vocab, _W)
    assert dim % _L == 0 and _W % _L == 0 and tail % 8 == 0
    nit = nblk // _NW + 2
    nit += nit % 2
    mesh = plsc.VectorSubcoreMesh(core_axis_name="c", subcore_axis_name="s")

    @functools.partial(
        pl.kernel,
        out_type=jax.ShapeDtypeStruct((vocab, _ROW), jnp.float32),
        mesh=mesh,
        scratch_types=[
            pltpu.VMEM((2, dim, _W), jnp.float32),
            pltpu.VMEM((2, _W, _ROW), jnp.float32),
            pltpu.VMEM((dim, tail), jnp.float32) if tail else None,
            pltpu.VMEM((tail, _ROW), jnp.float32) if tail else None,
            pltpu.SemaphoreType.DMA,
            pltpu.SemaphoreType.DMA,
        ],
        compiler_params=pltpu.CompilerParams(
            needs_layout_passes=False, disable_bounds_checks=True
        ),
    )
    def pad_kernel(embt_hbm, pad_hbm, src_v, tr_v, tsrc_v, ttr_v, lsem, wsem):
        wid = lax.axis_index("s") * _NC + lax.axis_index("c")
        iota = lax.iota(jnp.int32, _L)

        pltpu.async_copy(embt_hbm.at[:, pl.ds(wid * _W, _W)], src_v.at[0], lsem)

        @pl.loop(0, nit, step=2)
        def _blk(t0):
            for h in range(2):
                t = t0 + h
                b = wid + t * _NW

                @pl.when(b < nblk)
                def _():
                    v0 = b * _W

                    @pl.when(b + _NW < nblk)
                    def _():
                        pltpu.async_copy(
                            embt_hbm.at[:, pl.ds(v0 + _NW * _W, _W)],
                            src_v.at[1 - h],
                            lsem,
                        )

                    pltpu.make_async_copy(
                        embt_hbm.at[:, pl.ds(0, _W)], src_v.at[h], lsem
                    ).wait()

                    @pl.when(t0 > 0)
                    def _():
                        pltpu.make_async_copy(
                            tr_v.at[h], pad_hbm.at[pl.ds(0, _W)], wsem
                        ).wait()

                    src = src_v.at[h]
                    tr = tr_v.at[h]

                    @plsc.parallel_loop(0, _W // _L, unroll=2)
                    def _grp(j):
                        jv = j * _L + iota
                        for d0 in range(0, dim, 8):
                            dvs = [jnp.full((_L,), d, jnp.int32) for d in range(d0, d0 + 8)]
                            vs = [plsc.load_gather(src, [dv, jv]) for dv in dvs]
                            for dv, v in zip(dvs, vs):
                                plsc.store_scatter(tr, [jv, dv], v)

                    pltpu.async_copy(tr_v.at[h], pad_hbm.at[pl.ds(v0, _W)], wsem)

                if tail:

                    @pl.when(b == nblk)
                    def _():
                        v0 = nblk * _W
                        pltpu.sync_copy(embt_hbm.at[:, pl.ds(v0, tail)], tsrc_v)

                        @plsc.parallel_loop(0, tail // _L, unroll=2)
                        def _grp(j):
                            jv = j * _L + iota
                            for d0 in range(0, dim, 8):
                                dvs = [
                                    jnp.full((_L,), d, jnp.int32)
                                    for d in range(d0, d0 + 8)
                                ]
                                vs = [plsc.load_gather(tsrc_v, [dv, jv]) for dv in dvs]
                                for dv, v in zip(dvs, vs):
                                    plsc.store_scatter(ttr_v, [jv, dv], v)

                        pltpu.sync_copy(ttr_v, pad_hbm.at[pl.ds(v0, tail)])

        for h in range(2):
            pltpu.make_async_copy(tr_v.at[h], pad_hbm.at[pl.ds(0, _W)], wsem).wait()

    return pad_kernel


@functools.lru_cache(maxsize=None)
def _make_gather(vocab, dim, batch, seq):
    bw = batch // _NW
    assert batch % _NW == 0 and bw % 128 == 0 and bw <= 128
    assert dim % _L == 0 and seq % 2 == 0
    mesh = plsc.VectorSubcoreMesh(core_axis_name="c", subcore_axis_name="s")

    @functools.partial(
        pl.kernel,
        out_type=jax.ShapeDtypeStruct((seq, dim, batch), jnp.float32),
        mesh=mesh,
        scratch_types=[
            pltpu.VMEM((seq, bw), jnp.int32),
            pltpu.VMEM((2, bw, _ROW), jnp.float32),
            pltpu.VMEM((2, dim, bw), jnp.float32),
            pltpu.SemaphoreType.DMA,
            pltpu.SemaphoreType.DMA,
        ],
        compiler_params=pltpu.CompilerParams(
            needs_layout_passes=False, disable_bounds_checks=True
        ),
    )
    def gather_kernel(xt_hbm, table_hbm, out_hbm, idx_v, rows_v, tr_v, gsem, wsem):
        wid = lax.axis_index("s") * _NC + lax.axis_index("c")
        b0 = wid * bw
        pltpu.sync_copy(xt_hbm.at[:, pl.ds(b0, bw)], idx_v)
        iota = lax.iota(jnp.int32, _L)

        pltpu.async_copy(table_hbm.at[idx_v.at[0]], rows_v.at[0], gsem)

        @pl.loop(0, seq, step=2)
        def _task(t0):
            for h in range(2):
                t = t0 + h

                @pl.when(t + 1 < seq)
                def _():
                    pltpu.async_copy(
                        table_hbm.at[idx_v.at[t + 1]], rows_v.at[1 - h], gsem
                    )

                pltpu.make_async_copy(
                    table_hbm.at[idx_v.at[0]], rows_v.at[h], gsem
                ).wait()

                @pl.when(t0 > 0)
                def _():
                    pltpu.make_async_copy(
                        tr_v.at[h], out_hbm.at[0, :, pl.ds(0, bw)], wsem
                    ).wait()

                rows = rows_v.at[h]
                tr = tr_v.at[h]

                @plsc.parallel_loop(0, bw // _L, unroll=2)
                def _grp(g):
                    gv = g * _L + iota
                    for d0 in range(0, dim, 8):
                        vs = [
                            plsc.load_gather(rows, [gv, jnp.full((_L,), d, jnp.int32)])
                            for d in range(d0, d0 + 8)
                        ]
                        for k, v in enumerate(vs):
                            tr.at[d0 + k][pl.ds(g * _L, _L)] = v

                pltpu.async_copy(tr_v.at[h], out_hbm.at[t, :, pl.ds(b0, bw)], wsem)

        for h in range(2):
            pltpu.make_async_copy(
                tr_v.at[h], out_hbm.at[0, :, pl.ds(0, bw)], wsem
            ).wait()

    return gather_kernel


def kernel(x, embeddings):
    batch, seq = x.shape
    vocab, dim = embeddings.shape
    xt = jnp.transpose(x.astype(jnp.int32))
    embt = jnp.transpose(embeddings)
    padded = _make_pad(vocab, dim)(embt)
    out = _make_gather(vocab, dim, batch, seq)(xt, padded)
    return jnp.transpose(out, (2, 0, 1))


# speedup vs baseline: 1.7248x; 1.7248x over previous
import functools

import jax
import jax.numpy as jnp
from jax import lax
from jax.experimental import pallas as pl
from jax.experimental.pallas import tpu as pltpu
from jax.experimental.pallas import tpu_sc as plsc

_NC = 2
_NS = 16
_NW = _NC * _NS
_L = 16
_ROW = 128

_W = 256


@functools.lru_cache(maxsize=None)
def _make_pad(vocab, dim):
    nblk, tail = divmod(vocab, _W)
    assert dim % _L == 0 and _W % _L == 0 and tail % 8 == 0
    nit = nblk // _NW + 2
    nit += nit % 2
    mesh = plsc.VectorSubcoreMesh(core_axis_name="c", subcore_axis_name="s")

    @functools.partial(
        pl.kernel,
        out_type=jax.ShapeDtypeStruct((vocab, _ROW), jnp.float32),
        mesh=mesh,
        scratch_types=[
            pltpu.VMEM((2, dim, _W), jnp.float32),
            pltpu.VMEM((2, _W, _ROW), jnp.float32),
            pltpu.VMEM((dim, tail), jnp.float32) if tail else None,
            pltpu.VMEM((tail, _ROW), jnp.float32) if tail else None,
            pltpu.SemaphoreType.DMA,
            pltpu.SemaphoreType.DMA,
        ],
        compiler_params=pltpu.CompilerParams(
            needs_layout_passes=False, disable_bounds_checks=True
        ),
    )
    def pad_kernel(embt_hbm, pad_hbm, src_v, tr_v, tsrc_v, ttr_v, lsem, wsem):
        wid = lax.axis_index("s") * _NC + lax.axis_index("c")
        iota = lax.iota(jnp.int32, _L)

        pltpu.async_copy(embt_hbm.at[:, pl.ds(wid * _W, _W)], src_v.at[0], lsem)

        @pl.loop(0, nit, step=2)
        def _blk(t0):
            for h in range(2):
                t = t0 + h
                b = wid + t * _NW

                @pl.when(b < nblk)
                def _():
                    v0 = b * _W

                    @pl.when(b + _NW < nblk)
                    def _():
                        pltpu.async_copy(
                            embt_hbm.at[:, pl.ds(v0 + _NW * _W, _W)],
                            src_v.at[1 - h],
                            lsem,
                        )

                    pltpu.make_async_copy(
                        embt_hbm.at[:, pl.ds(0, _W)], src_v.at[h], lsem
                    ).wait()

                    @pl.when(t0 > 0)
                    def _():
                        pltpu.make_async_copy(
                            tr_v.at[h], pad_hbm.at[pl.ds(0, _W)], wsem
                        ).wait()

                    src = src_v.at[h]
                    tr = tr_v.at[h]

                    @plsc.parallel_loop(0, _W // _L, unroll=2)
                    def _grp(j):
                        jv = j * _L + iota
                        for d0 in range(0, dim, 8):
                            dvs = [jnp.full((_L,), d, jnp.int32) for d in range(d0, d0 + 8)]
                            vs = [plsc.load_gather(src, [dv, jv]) for dv in dvs]
                            for dv, v in zip(dvs, vs):
                                plsc.store_scatter(tr, [jv, dv], v)

                    pltpu.async_copy(tr_v.at[h], pad_hbm.at[pl.ds(v0, _W)], wsem)

                if tail:

                    @pl.when(b == nblk)
                    def _():
                        v0 = nblk * _W
                        pltpu.sync_copy(embt_hbm.at[:, pl.ds(v0, tail)], tsrc_v)

                        @plsc.parallel_loop(0, tail // _L, unroll=2)
                        def _grp(j):
                            jv = j * _L + iota
                            for d0 in range(0, dim, 8):
                                dvs = [
                                    jnp.full((_L,), d, jnp.int32)
                                    for d in range(d0, d0 + 8)
                                ]
                                vs = [plsc.load_gather(tsrc_v, [dv, jv]) for dv in dvs]
                                for dv, v in zip(dvs, vs):
                                    plsc.store_scatter(ttr_v, [jv, dv], v)

                        pltpu.sync_copy(ttr_v, pad_hbm.at[pl.ds(v0, tail)])

        for h in range(2):
            pltpu.make_async_copy(tr_v.at[h], pad_hbm.at[pl.ds(0, _W)], wsem).wait()

    return pad_kernel


@functools.lru_cache(maxsize=None)
def _make_gather(vocab, dim, batch, seq):
    bw = batch // _NW
    assert batch % _NW == 0 and bw % 128 == 0 and bw <= 128
    assert dim % _L == 0 and seq % 2 == 0
    mesh = plsc.VectorSubcoreMesh(core_axis_name="c", subcore_axis_name="s")

    @functools.partial(
        pl.kernel,
        out_type=jax.ShapeDtypeStruct((batch * seq, _ROW), jnp.float32),
        mesh=mesh,
        scratch_types=[
            pltpu.VMEM((batch * seq // _NW // 128, 128), jnp.int32),
            pltpu.VMEM((2, bw, _ROW), jnp.float32),
            pltpu.VMEM((2, dim, bw), jnp.float32),
            pltpu.SemaphoreType.DMA,
            pltpu.SemaphoreType.DMA,
        ],
        compiler_params=pltpu.CompilerParams(
            needs_layout_passes=False, disable_bounds_checks=True
        ),
    )
    def gather_kernel(x_hbm, table_hbm, out_hbm, idx_v, rows_v, tr_v, gsem, wsem):
        wid = lax.axis_index("s") * _NC + lax.axis_index("c")
        n_per_w = batch * seq // _NW
        nt = n_per_w // bw
        pltpu.sync_copy(x_hbm.at[pl.ds(wid * nt, nt)], idx_v)

        pltpu.async_copy(table_hbm.at[idx_v.at[0]], rows_v.at[0], gsem)

        @pl.loop(0, nt, step=2)
        def _task(t0):
            for h in range(2):
                t = t0 + h

                @pl.when(t > 0)
                def _():
                    pltpu.make_async_copy(
                        rows_v.at[1 - h], out_hbm.at[pl.ds(0, bw)], wsem
                    ).wait()

                @pl.when(t + 1 < nt)
                def _():
                    pltpu.async_copy(
                        table_hbm.at[idx_v.at[t + 1]], rows_v.at[1 - h], gsem
                    )

                pltpu.make_async_copy(
                    table_hbm.at[idx_v.at[0]], rows_v.at[h], gsem
                ).wait()

                pltpu.async_copy(
                    rows_v.at[h], out_hbm.at[pl.ds(wid * n_per_w + t * bw, bw)], wsem
                )

        pltpu.make_async_copy(rows_v.at[0], out_hbm.at[pl.ds(0, bw)], wsem).wait()

    return gather_kernel


def kernel(x, embeddings):
    batch, seq = x.shape
    vocab, dim = embeddings.shape
    padded = jnp.pad(embeddings, ((0, 0), (0, _ROW - dim)))
    idx = x.reshape(batch * seq).astype(jnp.int32)
    out2 = _make_gather(vocab, dim, batch, seq)(idx.reshape(batch * seq // 128, 128), padded)
    return out2[:, :dim].reshape(batch, seq, dim)
